# baseline (speedup 1.0000x reference)
import functools

import jax
import jax.numpy as jnp
from jax import lax
from jax.experimental import pallas as pl
from jax.experimental.pallas import tpu as pltpu
from jax.experimental.pallas import tpu_sc as plsc

_L = 16
_W = 32


def _make_emb_kernel(B, V, D, NC, NS):
    NW = NC * NS
    b_per_w = B // NW
    n_grp = b_per_w // _L

    mesh = plsc.VectorSubcoreMesh(core_axis_name="c", subcore_axis_name="s")

    @functools.partial(
        pl.kernel,
        out_type=jax.ShapeDtypeStruct((4, 8, B), jnp.float32),
        mesh=mesh,
        scratch_types=[
            pltpu.VMEM((b_per_w,), jnp.int32),
            pltpu.VMEM((b_per_w,), jnp.int32),
            pltpu.VMEM((b_per_w,), jnp.int32),
            pltpu.VMEM((2, D * _L, _W), jnp.float32),
            pltpu.VMEM((D, b_per_w), jnp.float32),
            pltpu.SemaphoreType.DMA,
            pltpu.SemaphoreType.DMA,
            pltpu.SemaphoreType.DMA,
        ],
        compiler_params=pltpu.CompilerParams(
            needs_layout_passes=False, use_tc_tiling_on_sc=False
        ),
    )
    def emb_kernel(
        idx_hbm, tab4, out3, idx_v, cid_v, sub_v, buf, gat_v, sem0, sem1, osem
    ):
        wid = lax.axis_index("s") * NC + lax.axis_index("c")
        pltpu.sync_copy(idx_hbm.at[wid], idx_v)
        for g in range(n_grp):
            iv = idx_v[pl.ds(g * _L, _L)]
            cid_v[pl.ds(g * _L, _L)] = lax.shift_right_logical(iv, 5)
            sub_v[pl.ds(g * _L, _L)] = lax.bitwise_and(iv, _W - 1)

        sems = (sem0, sem1)

        def fire(g, par):
            cvec = cid_v[pl.ds(g * _L, _L)]
            for d in range(D):
                pltpu.async_copy(
                    tab4.at[d // 8, d % 8].at[cvec],
                    buf.at[par, pl.ds(d * _L, _L)],
                    sems[par],
                )

        def drain_extract(g, par):
            pltpu.make_async_copy(
                tab4.at[0, 0, pl.ds(0, D * _L)],
                buf.at[par],
                sems[par],
            ).wait()
            sub = sub_v[pl.ds(g * _L, _L)]
            rvec = lax.iota(jnp.int32, _L)
            for d in range(D):
                vals = plsc.load_gather(
                    buf, [jnp.full((_L,), par, jnp.int32), rvec + d * _L, sub]
                )
                gat_v[d, pl.ds(g * _L, _L)] = vals

        fire(0, 0)

        def body(g, _):
            @pl.when(lax.rem(g, 2) == 0)
            def _():
                fire(g + 1, 1)
                drain_extract(g, 0)

            @pl.when(lax.rem(g, 2) == 1)
            def _():
                fire(g + 1, 0)
                drain_extract(g, 1)
            return ()

        lax.fori_loop(0, n_grp - 1, body, (), unroll=False)
        drain_extract(n_grp - 1, (n_grp - 1) % 2)

        writes = []
        for d in range(D):
            writes.append(
                pltpu.async_copy(
                    gat_v.at[d],
                    out3.at[d // 8, d % 8, pl.ds(wid * b_per_w, b_per_w)],
                    osem,
                )
            )
        for w in writes:
            w.wait()

    return emb_kernel


def kernel(cls, cls_emb):
    (B,) = cls.shape
    V, D = cls_emb.shape
    info = plsc.get_sparse_core_info()
    NC, NS = info.num_cores, info.num_subcores
    NW = NC * NS
    idx = cls.astype(jnp.int32).reshape(NW, B // NW)
    tab4 = cls_emb.T.reshape(4, 8, V // _W, _W)
    out3 = _make_emb_kernel(B, V, D, NC, NS)(idx, tab4)
    return out3.reshape(D, B).T
